# baseline (speedup 1.0000x reference)
import functools
import jax
import jax.numpy as jnp
import numpy as np
from jax import lax
from jax.experimental import pallas as pl
from jax.experimental.pallas import tpu as pltpu
from jax.experimental.pallas import tpu_sc as plsc

_K = 512
_N = 32768
_NVREG = _N // 16
_UNROLL = 8
_DUMMY = 2048


def _zero_hist(hist, nbins):
    z = jnp.zeros((16,), jnp.float32)

    def zbody(i, _):
        hist[pl.ds(i * 16, 16)] = z
        return 0

    lax.fori_loop(0, nbins // 16, zbody, 0)


def _hist_pass(x_v, hist, shift, nbits, pshift, prefix):
    ones = jnp.ones((16,), jnp.float32)
    lanemask = np.int32((1 << nbits) - 1)

    def body(i, _):
        for u in range(_UNROLL):
            v = x_v[pl.ds((i * _UNROLL + u) * 16, 16)]
            y = jnp.maximum(v, 0.0)
            k = lax.bitcast_convert_type(y, jnp.int32)
            idx = lax.shift_right_logical(k, shift) & lanemask
            sel = lax.shift_right_logical(k, pshift) == prefix
            idx = jnp.where(sel, idx, jnp.int32(_DUMMY))
            plsc.addupdate_scatter(hist, [idx], ones)
        return 0

    lax.fori_loop(0, _NVREG // _UNROLL, body, 0)


def _scan_level(hist, nbins, needed):
    def body(i, carry):
        total, found_v, before = carry
        vi = nbins // 16 - 1 - i
        v = hist[pl.ds(vi * 16, 16)]
        s = jnp.sum(v).astype(jnp.int32)
        new_total = total + s
        crossed = jnp.logical_and(new_total >= needed, found_v < 0)
        found_v = jnp.where(crossed, vi, found_v)
        before = jnp.where(crossed, total, before)
        return (new_total, found_v, before)

    _, fv, before = lax.fori_loop(
        0, nbins // 16, body,
        (jnp.int32(0), jnp.int32(-1), jnp.int32(0)),
    )
    v = hist[pl.ds(fv * 16, 16)].astype(jnp.int32)
    rv = lax.rev(v, (0,))
    cum = jnp.cumsum(rv)
    need_in = needed - before
    j = plsc.all_reduce_ffs(cum >= need_in)
    lane = lax.iota(jnp.int32, 16)
    cj = jnp.sum(jnp.where(lane == j, rv, 0))
    above = jnp.sum(jnp.where(lane == j, cum, 0)) - cj
    bucket = fv * 16 + 15 - j
    return bucket, need_in - above


def _sc_body(x_hbm, o_hbm, x_v, o_v, hist, sem):
    rows = x_hbm.shape[0]
    nw = 32
    rows_per_w = rows // nw
    wid = lax.axis_index("s") * 2 + lax.axis_index("c")

    for j in range(rows_per_w):
        r = wid * rows_per_w + j
        pltpu.sync_copy(x_hbm.at[r], x_v)

        _zero_hist(hist, 1024)
        _hist_pass(x_v, hist, 21, 10, 31, jnp.int32(0))
        b1, n1 = _scan_level(hist, 1024, jnp.int32(_K))

        _zero_hist(hist, 2048)
        _hist_pass(x_v, hist, 10, 11, 21, b1)
        b2, n2 = _scan_level(hist, 2048, n1)

        _zero_hist(hist, 1024)
        p2 = (b1 << 11) | b2
        _hist_pass(x_v, hist, 0, 10, 10, p2)
        b3, _ = _scan_level(hist, 1024, n2)

        tkey = (b1 << 21) | (b2 << 10) | b3

        def obody(i, _):
            for u in range(_UNROLL):
                sl = pl.ds((i * _UNROLL + u) * 16, 16)
                v = x_v[sl]
                y = jnp.maximum(v, 0.0)
                k = lax.bitcast_convert_type(y, jnp.int32)
                o_v[sl] = jnp.where(k >= tkey, y, 0.0)
            return 0

        lax.fori_loop(0, _NVREG // _UNROLL, obody, 0)
        pltpu.sync_copy(o_v, o_hbm.at[r])


def sc_topk(x):
    rows = x.shape[0]
    mesh = plsc.VectorSubcoreMesh(core_axis_name="c", subcore_axis_name="s")
    k = pl.kernel(
        _sc_body,
        out_type=jax.ShapeDtypeStruct((rows, _N), jnp.float32),
        mesh=mesh,
        compiler_params=pltpu.CompilerParams(needs_layout_passes=False),
        scratch_types=[
            pltpu.VMEM((_N,), jnp.float32),
            pltpu.VMEM((_N,), jnp.float32),
            pltpu.VMEM((2064,), jnp.float32),
            pltpu.SemaphoreType.DMA,
        ],
    )
    return k(x)


def kernel(x):
    return sc_topk(x)


# speedup vs baseline: 8.6855x; 8.6855x over previous
import jax
import jax.numpy as jnp
import numpy as np
from jax.experimental import pallas as pl
from jax.experimental.pallas import tpu as pltpu

_K = 512


def _topk_mask_kernel(x_ref, o_ref, key_ref):
    x = x_ref[...]
    key_ref[...] = jax.lax.bitcast_convert_type(jnp.maximum(x, 0.0), jnp.int32)
    rows, n = x.shape
    n_chunks = n // 128

    def body(i, prefix):
        shift = 30 - i
        cand = prefix + jnp.left_shift(np.int32(1), shift)
        accs = []
        for g in range(16):
            acc = None
            for j in range(n_chunks // 16):
                c = g * (n_chunks // 16) + j
                s = key_ref[:, c * 128:(c + 1) * 128]
                t = jnp.where(s >= cand, 1.0, 0.0)
                acc = t if acc is None else acc + t
            accs.append(acc)
        while len(accs) > 1:
            accs = [accs[k] + accs[k + 1] for k in range(0, len(accs), 2)]
        cnt = jnp.sum(accs[0], axis=1, keepdims=True)
        return jnp.where(cnt >= _K, cand, prefix)

    thresh = jax.lax.fori_loop(
        0, 31, body, jnp.zeros((rows, 1), jnp.int32)
    )
    key = key_ref[...]
    y = jax.lax.bitcast_convert_type(key, jnp.float32)
    o_ref[...] = jnp.where(key >= thresh, y, 0.0)


def kernel(x):
    m, n = x.shape
    block_rows = 64
    return pl.pallas_call(
        _topk_mask_kernel,
        grid=(m // block_rows,),
        in_specs=[pl.BlockSpec((block_rows, n), lambda i: (i, 0))],
        out_specs=pl.BlockSpec((block_rows, n), lambda i: (i, 0)),
        out_shape=jax.ShapeDtypeStruct((m, n), x.dtype),
        scratch_shapes=[pltpu.VMEM((block_rows, n), jnp.int32)],
    )(x)
